# Initial kernel scaffold; baseline (speedup 1.0000x reference)
import jax
import jax.numpy as jnp
from jax.experimental import pallas as pl


def kernel(x, keys, v):
    raise NotImplementedError("write your pallas kernel here")


# speedup vs baseline: 20.6070x; 20.6070x over previous
import functools

import jax
import jax.numpy as jnp
from jax import lax
from jax.experimental import pallas as pl
from jax.experimental.pallas import tpu as pltpu

N_EXPERTS = 8
D_INPUT = 1024
M = 2048
TOPK = 8
THETA = 0.5
BATCH = 1024

B_TILE = 256


def _expert_body(x_ref, keys_ref, masked_ref, fam_ref, y_ref, g_ref):
    x = x_ref[...]
    keys = keys_ref[0]
    xn = x / (jnp.sqrt(jnp.sum(x * x, axis=1, keepdims=True)) + 1e-9)
    kn = keys / (jnp.sqrt(jnp.sum(keys * keys, axis=1, keepdims=True)) + 1e-9)
    scores = lax.dot_general(
        xn.astype(jnp.bfloat16), kn.astype(jnp.bfloat16),
        (((1,), (1,)), ((), ())),
        preferred_element_type=jnp.float32,
    )

    s = scores
    tv = []
    for i in range(TOPK):
        m = jnp.max(s, axis=1, keepdims=True)
        tv.append(m)
        if i < TOPK - 1:
            s = jnp.where(s == m, -jnp.inf, s)

    kth = tv[-1]
    fam = sum(tv) / TOPK
    exps = [jnp.exp(t - tv[0]) for t in tv]
    z = sum(exps)
    y = sum(e * t for e, t in zip(exps, tv)) / z
    g = (fam > THETA).astype(jnp.float32)

    masked_ref[0] = jnp.where(scores >= kth, scores, -jnp.inf)
    fam_ref[0] = fam
    y_ref[0] = y
    g_ref[0] = g


def _route_body(fam_ref, y_ref, g_ref, w_ref, mf_ref, yo_ref, go_ref):
    wmax = fam_ref[0:1, :]
    widx = jnp.zeros((1, BATCH), dtype=jnp.int32)
    ysel = y_ref[0:1, :]
    gsel = g_ref[0:1, :]
    for e in range(1, N_EXPERTS):
        f = fam_ref[e:e + 1, :]
        m = f > wmax
        wmax = jnp.where(m, f, wmax)
        widx = jnp.where(m, e, widx)
        ysel = jnp.where(m, y_ref[e:e + 1, :], ysel)
        gsel = jnp.where(m, g_ref[e:e + 1, :], gsel)
    w_ref[...] = widx
    mf_ref[...] = wmax
    yo_ref[...] = ysel
    go_ref[...] = gsel


@jax.jit
def kernel(x, keys, v):
    n_btiles = BATCH // B_TILE
    masked, fam, y_e, g_e = pl.pallas_call(
        _expert_body,
        grid=(N_EXPERTS, n_btiles),
        in_specs=[
            pl.BlockSpec((B_TILE, D_INPUT), lambda e, b: (b, 0)),
            pl.BlockSpec((1, M, D_INPUT), lambda e, b: (e, 0, 0)),
        ],
        out_specs=[
            pl.BlockSpec((1, B_TILE, M), lambda e, b: (e, b, 0)),
            pl.BlockSpec((1, B_TILE, 1), lambda e, b: (e, b, 0)),
            pl.BlockSpec((1, B_TILE, 1), lambda e, b: (e, b, 0)),
            pl.BlockSpec((1, B_TILE, 1), lambda e, b: (e, b, 0)),
        ],
        out_shape=[
            jax.ShapeDtypeStruct((N_EXPERTS, BATCH, M), jnp.float32),
            jax.ShapeDtypeStruct((N_EXPERTS, BATCH, 1), jnp.float32),
            jax.ShapeDtypeStruct((N_EXPERTS, BATCH, 1), jnp.float32),
            jax.ShapeDtypeStruct((N_EXPERTS, BATCH, 1), jnp.float32),
        ],
    )(x, keys)
    fam = fam.reshape(N_EXPERTS, BATCH)
    y_e = y_e.reshape(N_EXPERTS, BATCH)
    g_e = g_e.reshape(N_EXPERTS, BATCH)

    winner, max_fam, y, g = pl.pallas_call(
        _route_body,
        out_shape=[
            jax.ShapeDtypeStruct((1, BATCH), jnp.int32),
            jax.ShapeDtypeStruct((1, BATCH), jnp.float32),
            jax.ShapeDtypeStruct((1, BATCH), jnp.float32),
            jax.ShapeDtypeStruct((1, BATCH), jnp.float32),
        ],
    )(fam, y_e, g_e)

    return (winner.reshape(BATCH), max_fam.reshape(BATCH),
            y.reshape(BATCH), g.reshape(BATCH), masked)


# speedup vs baseline: 24.5012x; 1.1890x over previous
import functools

import jax
import jax.numpy as jnp
from jax import lax
from jax.experimental import pallas as pl
from jax.experimental.pallas import tpu as pltpu

N_EXPERTS = 8
D_INPUT = 1024
M = 2048
TOPK = 8
THETA = 0.5
BATCH = 1024

B_TILE = 512


def _expert_body(x_ref, keys_ref, masked_ref, fam_ref, y_ref, g_ref,
                 xn_ref, kn_ref):
    e = pl.program_id(0)
    b = pl.program_id(1)

    @pl.when(e == 0)
    def _():
        xblk = x_ref[pl.ds(b * B_TILE, B_TILE), :]
        nrm = jnp.sqrt(jnp.sum(xblk * xblk, axis=1, keepdims=True)) + 1e-9
        xn_ref[pl.ds(b * B_TILE, B_TILE), :] = (xblk / nrm).astype(jnp.bfloat16)

    @pl.when(b == 0)
    def _():
        keys = keys_ref[0]
        nrm = jnp.sqrt(jnp.sum(keys * keys, axis=1, keepdims=True)) + 1e-9
        kn_ref[...] = (keys / nrm).astype(jnp.bfloat16)

    xn = xn_ref[pl.ds(b * B_TILE, B_TILE), :]
    scores = lax.dot_general(
        xn, kn_ref[...], (((1,), (1,)), ((), ())),
        preferred_element_type=jnp.float32,
    )

    s = scores
    tv = []
    for i in range(TOPK):
        m = jnp.max(s, axis=1, keepdims=True)
        tv.append(m)
        if i < TOPK - 1:
            s = jnp.where(s == m, -jnp.inf, s)

    kth = tv[-1]
    fam = sum(tv) / TOPK
    exps = [jnp.exp(t - tv[0]) for t in tv]
    z = sum(exps)
    y = sum(ev * t for ev, t in zip(exps, tv)) / z
    g = (fam > THETA).astype(jnp.float32)

    masked_ref[0] = jnp.where(scores >= kth, scores, -jnp.inf)
    fam_ref[0] = fam
    y_ref[0] = y
    g_ref[0] = g


def _route_body(fam_ref, y_ref, g_ref, w_ref, mf_ref, yo_ref, go_ref):
    wmax = fam_ref[0:1, :]
    widx = jnp.zeros((1, BATCH), dtype=jnp.int32)
    ysel = y_ref[0:1, :]
    gsel = g_ref[0:1, :]
    for e in range(1, N_EXPERTS):
        f = fam_ref[e:e + 1, :]
        m = f > wmax
        wmax = jnp.where(m, f, wmax)
        widx = jnp.where(m, e, widx)
        ysel = jnp.where(m, y_ref[e:e + 1, :], ysel)
        gsel = jnp.where(m, g_ref[e:e + 1, :], gsel)
    w_ref[...] = widx
    mf_ref[...] = wmax
    yo_ref[...] = ysel
    go_ref[...] = gsel


@jax.jit
def kernel(x, keys, v):
    n_btiles = BATCH // B_TILE
    masked, fam, y_e, g_e = pl.pallas_call(
        _expert_body,
        grid=(N_EXPERTS, n_btiles),
        in_specs=[
            pl.BlockSpec((BATCH, D_INPUT), lambda e, b: (0, 0)),
            pl.BlockSpec((1, M, D_INPUT), lambda e, b: (e, 0, 0)),
        ],
        out_specs=[
            pl.BlockSpec((1, B_TILE, M), lambda e, b: (e, b, 0)),
            pl.BlockSpec((1, B_TILE, 1), lambda e, b: (e, b, 0)),
            pl.BlockSpec((1, B_TILE, 1), lambda e, b: (e, b, 0)),
            pl.BlockSpec((1, B_TILE, 1), lambda e, b: (e, b, 0)),
        ],
        out_shape=[
            jax.ShapeDtypeStruct((N_EXPERTS, BATCH, M), jnp.float32),
            jax.ShapeDtypeStruct((N_EXPERTS, BATCH, 1), jnp.float32),
            jax.ShapeDtypeStruct((N_EXPERTS, BATCH, 1), jnp.float32),
            jax.ShapeDtypeStruct((N_EXPERTS, BATCH, 1), jnp.float32),
        ],
        scratch_shapes=[
            pltpu.VMEM((BATCH, D_INPUT), jnp.bfloat16),
            pltpu.VMEM((M, D_INPUT), jnp.bfloat16),
        ],
    )(x, keys)
    fam = fam.reshape(N_EXPERTS, BATCH)
    y_e = y_e.reshape(N_EXPERTS, BATCH)
    g_e = g_e.reshape(N_EXPERTS, BATCH)

    winner, max_fam, y, g = pl.pallas_call(
        _route_body,
        out_shape=[
            jax.ShapeDtypeStruct((1, BATCH), jnp.int32),
            jax.ShapeDtypeStruct((1, BATCH), jnp.float32),
            jax.ShapeDtypeStruct((1, BATCH), jnp.float32),
            jax.ShapeDtypeStruct((1, BATCH), jnp.float32),
        ],
    )(fam, y_e, g_e)

    return (winner.reshape(BATCH), max_fam.reshape(BATCH),
            y.reshape(BATCH), g.reshape(BATCH), masked)


# speedup vs baseline: 28.8147x; 1.1761x over previous
import functools

import jax
import jax.numpy as jnp
from jax import lax
from jax.experimental import pallas as pl
from jax.experimental.pallas import tpu as pltpu

N_EXPERTS = 8
D_INPUT = 1024
M = 2048
TOPK = 8
THETA = 0.5
BATCH = 1024

B_TILE = 512


def _expert_body(x_ref, keys_ref, masked_ref, fam_ref, y_ref, g_ref,
                 xn_ref, kn_ref):
    e = pl.program_id(0)
    b = pl.program_id(1)

    @pl.when(e == 0)
    def _():
        xblk = x_ref[pl.ds(b * B_TILE, B_TILE), :]
        nrm = jnp.sqrt(jnp.sum(xblk * xblk, axis=1, keepdims=True)) + 1e-9
        xn_ref[pl.ds(b * B_TILE, B_TILE), :] = (xblk / nrm).astype(jnp.bfloat16)

    @pl.when(b == 0)
    def _():
        keys = keys_ref[0]
        nrm = jnp.sqrt(jnp.sum(keys * keys, axis=1, keepdims=True)) + 1e-9
        kn_ref[...] = (keys / nrm).astype(jnp.bfloat16)

    xn = xn_ref[pl.ds(b * B_TILE, B_TILE), :]
    halves = [
        lax.dot_general(
            xn, kn_ref[pl.ds(h * (M // 2), M // 2), :],
            (((1,), (1,)), ((), ())),
            preferred_element_type=jnp.float32,
        )
        for h in range(2)
    ]

    CH, CW = 16, M // 16
    SUB = 64

    def _sortnet(v, pairs):
        v = list(v)
        for i, j in pairs:
            hi = jnp.maximum(v[i], v[j])
            lo = jnp.minimum(v[i], v[j])
            v[i], v[j] = hi, lo
        return v

    _S8 = [(0, 1), (2, 3), (4, 5), (6, 7),
           (0, 2), (1, 3), (4, 6), (5, 7),
           (1, 2), (5, 6),
           (0, 4), (1, 5), (2, 6), (3, 7),
           (2, 4), (3, 5),
           (1, 2), (3, 4), (5, 6)]
    _B8 = [(0, 4), (1, 5), (2, 6), (3, 7),
           (0, 2), (1, 3), (4, 6), (5, 7),
           (0, 1), (2, 3), (4, 5), (6, 7)]

    tv_parts = [[] for _ in range(TOPK)]
    for rt in range(B_TILE // SUB):
        r0 = rt * SUB
        ch = [halves[c // 8][r0:r0 + SUB, (c % 8) * CW:(c % 8 + 1) * CW]
              for c in range(CH)]
        s1 = _sortnet(ch[:8], _S8)
        s2 = _sortnet(ch[8:], _S8)
        bit = [jnp.maximum(s1[i], s2[7 - i]) for i in range(8)]
        stk = _sortnet(bit, _B8)
        for i in range(TOPK):
            m = jnp.max(stk[0], axis=1, keepdims=True)
            tv_parts[i].append(m)
            if i < TOPK - 1:
                cond = stk[0] == m
                for d in range(7 - i):
                    stk[d] = jnp.where(cond, stk[d + 1], stk[d])

    tv = [jnp.concatenate(p, axis=0) for p in tv_parts]
    kth = tv[-1]
    fam = sum(tv) / TOPK
    exps = [jnp.exp(t - tv[0]) for t in tv]
    z = sum(exps)
    y = sum(ev * t for ev, t in zip(exps, tv)) / z
    g = (fam > THETA).astype(jnp.float32)

    for h in range(2):
        masked_ref[0, :, pl.ds(h * (M // 2), M // 2)] = jnp.where(
            halves[h] >= kth, halves[h], -jnp.inf)
    fam_ref[0] = fam
    y_ref[0] = y
    g_ref[0] = g


def _route_body(fam_ref, y_ref, g_ref, w_ref, mf_ref, yo_ref, go_ref):
    wmax = fam_ref[0:1, :]
    widx = jnp.zeros((1, BATCH), dtype=jnp.int32)
    ysel = y_ref[0:1, :]
    gsel = g_ref[0:1, :]
    for e in range(1, N_EXPERTS):
        f = fam_ref[e:e + 1, :]
        m = f > wmax
        wmax = jnp.where(m, f, wmax)
        widx = jnp.where(m, e, widx)
        ysel = jnp.where(m, y_ref[e:e + 1, :], ysel)
        gsel = jnp.where(m, g_ref[e:e + 1, :], gsel)
    w_ref[...] = widx
    mf_ref[...] = wmax
    yo_ref[...] = ysel
    go_ref[...] = gsel


@jax.jit
def kernel(x, keys, v):
    n_btiles = BATCH // B_TILE
    masked, fam, y_e, g_e = pl.pallas_call(
        _expert_body,
        grid=(N_EXPERTS, n_btiles),
        in_specs=[
            pl.BlockSpec((BATCH, D_INPUT), lambda e, b: (0, 0)),
            pl.BlockSpec((1, M, D_INPUT), lambda e, b: (e, 0, 0)),
        ],
        out_specs=[
            pl.BlockSpec((1, B_TILE, M), lambda e, b: (e, b, 0)),
            pl.BlockSpec((1, B_TILE, 1), lambda e, b: (e, b, 0)),
            pl.BlockSpec((1, B_TILE, 1), lambda e, b: (e, b, 0)),
            pl.BlockSpec((1, B_TILE, 1), lambda e, b: (e, b, 0)),
        ],
        out_shape=[
            jax.ShapeDtypeStruct((N_EXPERTS, BATCH, M), jnp.float32),
            jax.ShapeDtypeStruct((N_EXPERTS, BATCH, 1), jnp.float32),
            jax.ShapeDtypeStruct((N_EXPERTS, BATCH, 1), jnp.float32),
            jax.ShapeDtypeStruct((N_EXPERTS, BATCH, 1), jnp.float32),
        ],
        scratch_shapes=[
            pltpu.VMEM((BATCH, D_INPUT), jnp.bfloat16),
            pltpu.VMEM((M, D_INPUT), jnp.bfloat16),
        ],
    )(x, keys)
    fam = fam.reshape(N_EXPERTS, BATCH)
    y_e = y_e.reshape(N_EXPERTS, BATCH)
    g_e = g_e.reshape(N_EXPERTS, BATCH)

    winner, max_fam, y, g = pl.pallas_call(
        _route_body,
        out_shape=[
            jax.ShapeDtypeStruct((1, BATCH), jnp.int32),
            jax.ShapeDtypeStruct((1, BATCH), jnp.float32),
            jax.ShapeDtypeStruct((1, BATCH), jnp.float32),
            jax.ShapeDtypeStruct((1, BATCH), jnp.float32),
        ],
    )(fam, y_e, g_e)

    return (winner.reshape(BATCH), max_fam.reshape(BATCH),
            y.reshape(BATCH), g.reshape(BATCH), masked)


# speedup vs baseline: 29.8584x; 1.0362x over previous
import functools

import jax
import jax.numpy as jnp
from jax import lax
from jax.experimental import pallas as pl
from jax.experimental.pallas import tpu as pltpu

N_EXPERTS = 8
D_INPUT = 1024
M = 2048
TOPK = 8
THETA = 0.5
BATCH = 1024

B_TILE = 512


def _expert_body(x_ref, keys_ref, masked_ref, w_ref, mf_ref, yo_ref, go_ref,
                 xn_ref, kn_ref):
    e = pl.program_id(0)
    b = pl.program_id(1)

    @pl.when(e == 0)
    def _():
        xblk = x_ref[pl.ds(b * B_TILE, B_TILE), :]
        nrm = jnp.sqrt(jnp.sum(xblk * xblk, axis=1, keepdims=True)) + 1e-9
        xn_ref[pl.ds(b * B_TILE, B_TILE), :] = (xblk / nrm).astype(jnp.bfloat16)

    @pl.when(b == 0)
    def _():
        keys = keys_ref[0]
        nrm = jnp.sqrt(jnp.sum(keys * keys, axis=1, keepdims=True)) + 1e-9
        kn_ref[...] = (keys / nrm).astype(jnp.bfloat16)

    xn = xn_ref[pl.ds(b * B_TILE, B_TILE), :]
    halves = [
        lax.dot_general(
            xn, kn_ref[pl.ds(h * (M // 2), M // 2), :],
            (((1,), (1,)), ((), ())),
            preferred_element_type=jnp.float32,
        )
        for h in range(2)
    ]

    CH, CW = 16, M // 16
    SUB = 64

    def _sortnet(v, pairs):
        v = list(v)
        for i, j in pairs:
            hi = jnp.maximum(v[i], v[j])
            lo = jnp.minimum(v[i], v[j])
            v[i], v[j] = hi, lo
        return v

    _S8 = [(0, 1), (2, 3), (4, 5), (6, 7),
           (0, 2), (1, 3), (4, 6), (5, 7),
           (1, 2), (5, 6),
           (0, 4), (1, 5), (2, 6), (3, 7),
           (2, 4), (3, 5),
           (1, 2), (3, 4), (5, 6)]
    _B8 = [(0, 4), (1, 5), (2, 6), (3, 7),
           (0, 2), (1, 3), (4, 6), (5, 7),
           (0, 1), (2, 3), (4, 5), (6, 7)]

    tv_parts = [[] for _ in range(TOPK)]
    for rt in range(B_TILE // SUB):
        r0 = rt * SUB
        ch = [halves[c // 8][r0:r0 + SUB, (c % 8) * CW:(c % 8 + 1) * CW]
              for c in range(CH)]
        s1 = _sortnet(ch[:8], _S8)
        s2 = _sortnet(ch[8:], _S8)
        bit = [jnp.maximum(s1[i], s2[7 - i]) for i in range(8)]
        stk = _sortnet(bit, _B8)
        for i in range(TOPK):
            m = jnp.max(stk[0], axis=1, keepdims=True)
            tv_parts[i].append(m)
            if i < TOPK - 1:
                cond = stk[0] == m
                for d in range(7 - i):
                    stk[d] = jnp.where(cond, stk[d + 1], stk[d])

    tv = [jnp.concatenate(p, axis=0) for p in tv_parts]
    kth = tv[-1]
    fam = sum(tv) / TOPK
    exps = [jnp.exp(t - tv[0]) for t in tv]
    z = sum(exps)
    y = sum(ev * t for ev, t in zip(exps, tv)) / z
    g = (fam > THETA).astype(jnp.float32)

    for h in range(2):
        masked_ref[0, :, pl.ds(h * (M // 2), M // 2)] = jnp.where(
            halves[h] >= kth, halves[h], -jnp.inf)

    sl = pl.ds(b * B_TILE, B_TILE)

    @pl.when(e == 0)
    def _():
        mf_ref[sl] = fam
        w_ref[sl] = jnp.zeros((B_TILE, 1), jnp.int32)
        yo_ref[sl] = y
        go_ref[sl] = g

    @pl.when(e > 0)
    def _():
        wm = mf_ref[sl]
        cond = fam > wm
        mf_ref[sl] = jnp.where(cond, fam, wm)
        w_ref[sl] = jnp.where(cond, e, w_ref[sl])
        yo_ref[sl] = jnp.where(cond, y, yo_ref[sl])
        go_ref[sl] = jnp.where(cond, g, go_ref[sl])


@jax.jit
def kernel(x, keys, v):
    n_btiles = BATCH // B_TILE
    masked, winner, max_fam, y, g = pl.pallas_call(
        _expert_body,
        grid=(N_EXPERTS, n_btiles),
        in_specs=[
            pl.BlockSpec((BATCH, D_INPUT), lambda e, b: (0, 0)),
            pl.BlockSpec((1, M, D_INPUT), lambda e, b: (e, 0, 0)),
        ],
        out_specs=[
            pl.BlockSpec((1, B_TILE, M), lambda e, b: (e, b, 0)),
            pl.BlockSpec((BATCH, 1), lambda e, b: (0, 0)),
            pl.BlockSpec((BATCH, 1), lambda e, b: (0, 0)),
            pl.BlockSpec((BATCH, 1), lambda e, b: (0, 0)),
            pl.BlockSpec((BATCH, 1), lambda e, b: (0, 0)),
        ],
        out_shape=[
            jax.ShapeDtypeStruct((N_EXPERTS, BATCH, M), jnp.float32),
            jax.ShapeDtypeStruct((BATCH, 1), jnp.int32),
            jax.ShapeDtypeStruct((BATCH, 1), jnp.float32),
            jax.ShapeDtypeStruct((BATCH, 1), jnp.float32),
            jax.ShapeDtypeStruct((BATCH, 1), jnp.float32),
        ],
        scratch_shapes=[
            pltpu.VMEM((BATCH, D_INPUT), jnp.bfloat16),
            pltpu.VMEM((M, D_INPUT), jnp.bfloat16),
        ],
    )(x, keys)

    return (winner.reshape(BATCH), max_fam.reshape(BATCH),
            y.reshape(BATCH), g.reshape(BATCH), masked)


# speedup vs baseline: 33.0464x; 1.1068x over previous
import functools

import jax
import jax.numpy as jnp
from jax import lax
from jax.experimental import pallas as pl
from jax.experimental.pallas import tpu as pltpu

N_EXPERTS = 8
D_INPUT = 1024
M = 2048
TOPK = 8
THETA = 0.5
BATCH = 1024

B_TILE = 1024


def _expert_body(x_ref, keys_ref, masked_ref, w_ref, mf_ref, yo_ref, go_ref,
                 xn_ref, kn_ref):
    e = pl.program_id(0)
    b = pl.program_id(1)

    @pl.when(e == 0)
    def _():
        xblk = x_ref[pl.ds(b * B_TILE, B_TILE), :]
        nrm = jnp.sqrt(jnp.sum(xblk * xblk, axis=1, keepdims=True)) + 1e-9
        xn_ref[pl.ds(b * B_TILE, B_TILE), :] = (xblk / nrm).astype(jnp.bfloat16)

    @pl.when(b == 0)
    def _():
        keys = keys_ref[0]
        nrm = jnp.sqrt(jnp.sum(keys * keys, axis=1, keepdims=True)) + 1e-9
        kn_ref[...] = (keys / nrm).astype(jnp.bfloat16)

    xn = xn_ref[pl.ds(b * B_TILE, B_TILE), :]
    halves = [
        lax.dot_general(
            xn, kn_ref[pl.ds(h * (M // 2), M // 2), :],
            (((1,), (1,)), ((), ())),
            preferred_element_type=jnp.float32,
        )
        for h in range(2)
    ]

    CH, CW = 16, M // 16
    SUB = 64

    def _sortnet(v, pairs):
        v = list(v)
        for i, j in pairs:
            hi = jnp.maximum(v[i], v[j])
            lo = jnp.minimum(v[i], v[j])
            v[i], v[j] = hi, lo
        return v

    _S8 = [(0, 1), (2, 3), (4, 5), (6, 7),
           (0, 2), (1, 3), (4, 6), (5, 7),
           (1, 2), (5, 6),
           (0, 4), (1, 5), (2, 6), (3, 7),
           (2, 4), (3, 5),
           (1, 2), (3, 4), (5, 6)]
    _B8 = [(0, 4), (1, 5), (2, 6), (3, 7),
           (0, 2), (1, 3), (4, 6), (5, 7),
           (0, 1), (2, 3), (4, 5), (6, 7)]

    tv_parts = [[] for _ in range(TOPK)]
    for rt in range(B_TILE // SUB):
        r0 = rt * SUB
        ch = [halves[c // 8][r0:r0 + SUB, (c % 8) * CW:(c % 8 + 1) * CW]
              for c in range(CH)]
        s1 = _sortnet(ch[:8], _S8)
        s2 = _sortnet(ch[8:], _S8)
        bit = [jnp.maximum(s1[i], s2[7 - i]) for i in range(8)]
        stk = _sortnet(bit, _B8)
        for i in range(TOPK):
            m = jnp.max(stk[0], axis=1, keepdims=True)
            tv_parts[i].append(m)
            if i < TOPK - 1:
                cond = stk[0] == m
                for d in range(7 - i):
                    stk[d] = jnp.where(cond, stk[d + 1], stk[d])

    tv = [jnp.concatenate(p, axis=0) for p in tv_parts]
    kth = tv[-1]
    fam = sum(tv) / TOPK
    exps = [jnp.exp(t - tv[0]) for t in tv]
    z = sum(exps)
    y = sum(ev * t for ev, t in zip(exps, tv)) / z
    g = (fam > THETA).astype(jnp.float32)

    for h in range(2):
        masked_ref[0, :, pl.ds(h * (M // 2), M // 2)] = jnp.where(
            halves[h] >= kth, halves[h], -jnp.inf)

    sl = pl.ds(b * B_TILE, B_TILE)

    @pl.when(e == 0)
    def _():
        mf_ref[sl] = fam
        w_ref[sl] = jnp.zeros((B_TILE, 1), jnp.int32)
        yo_ref[sl] = y
        go_ref[sl] = g

    @pl.when(e > 0)
    def _():
        wm = mf_ref[sl]
        cond = fam > wm
        mf_ref[sl] = jnp.where(cond, fam, wm)
        w_ref[sl] = jnp.where(cond, e, w_ref[sl])
        yo_ref[sl] = jnp.where(cond, y, yo_ref[sl])
        go_ref[sl] = jnp.where(cond, g, go_ref[sl])


@jax.jit
def kernel(x, keys, v):
    n_btiles = BATCH // B_TILE
    masked, winner, max_fam, y, g = pl.pallas_call(
        _expert_body,
        grid=(N_EXPERTS, n_btiles),
        in_specs=[
            pl.BlockSpec((BATCH, D_INPUT), lambda e, b: (0, 0)),
            pl.BlockSpec((1, M, D_INPUT), lambda e, b: (e, 0, 0)),
        ],
        out_specs=[
            pl.BlockSpec((1, B_TILE, M), lambda e, b: (e, b, 0)),
            pl.BlockSpec((BATCH, 1), lambda e, b: (0, 0)),
            pl.BlockSpec((BATCH, 1), lambda e, b: (0, 0)),
            pl.BlockSpec((BATCH, 1), lambda e, b: (0, 0)),
            pl.BlockSpec((BATCH, 1), lambda e, b: (0, 0)),
        ],
        out_shape=[
            jax.ShapeDtypeStruct((N_EXPERTS, BATCH, M), jnp.float32),
            jax.ShapeDtypeStruct((BATCH, 1), jnp.int32),
            jax.ShapeDtypeStruct((BATCH, 1), jnp.float32),
            jax.ShapeDtypeStruct((BATCH, 1), jnp.float32),
            jax.ShapeDtypeStruct((BATCH, 1), jnp.float32),
        ],
        scratch_shapes=[
            pltpu.VMEM((BATCH, D_INPUT), jnp.bfloat16),
            pltpu.VMEM((M, D_INPUT), jnp.bfloat16),
        ],
    )(x, keys)

    return (winner.reshape(BATCH), max_fam.reshape(BATCH),
            y.reshape(BATCH), g.reshape(BATCH), masked)


# speedup vs baseline: 33.0770x; 1.0009x over previous
import functools

import jax
import jax.numpy as jnp
from jax import lax
from jax.experimental import pallas as pl
from jax.experimental.pallas import tpu as pltpu

N_EXPERTS = 8
D_INPUT = 1024
M = 2048
TOPK = 8
THETA = 0.5
BATCH = 1024

B_TILE = 1024


def _expert_body(x_ref, keys_ref, masked_ref, w_ref, mf_ref, yo_ref, go_ref,
                 xn_ref, kn_ref):
    e = pl.program_id(0)

    @pl.when(e == 0)
    def _():
        xblk = x_ref[...]
        nrm = jnp.sqrt(jnp.sum(xblk * xblk, axis=1, keepdims=True)) + 1e-9
        xn_ref[...] = (xblk / nrm).astype(jnp.bfloat16)

    keys = keys_ref[0]
    nrm = jnp.sqrt(jnp.sum(keys * keys, axis=1, keepdims=True)) + 1e-9
    kn_ref[...] = (keys / nrm).astype(jnp.bfloat16)

    xn = xn_ref[...]
    halves = [
        lax.dot_general(
            xn, kn_ref[pl.ds(h * (M // 2), M // 2), :],
            (((1,), (1,)), ((), ())),
            preferred_element_type=jnp.float32,
        )
        for h in range(2)
    ]

    CH, CW = 16, M // 16
    SUB = 64

    def _sortnet(v, pairs):
        v = list(v)
        for i, j in pairs:
            hi = jnp.maximum(v[i], v[j])
            lo = jnp.minimum(v[i], v[j])
            v[i], v[j] = hi, lo
        return v

    _S8 = [(0, 1), (2, 3), (4, 5), (6, 7),
           (0, 2), (1, 3), (4, 6), (5, 7),
           (1, 2), (5, 6),
           (0, 4), (1, 5), (2, 6), (3, 7),
           (2, 4), (3, 5),
           (1, 2), (3, 4), (5, 6)]
    _B8 = [(0, 4), (1, 5), (2, 6), (3, 7),
           (0, 2), (1, 3), (4, 6), (5, 7),
           (0, 1), (2, 3), (4, 5), (6, 7)]

    tv_parts = [[] for _ in range(TOPK)]
    for rt in range(B_TILE // SUB):
        r0 = rt * SUB
        ch = [halves[c // 8][r0:r0 + SUB, (c % 8) * CW:(c % 8 + 1) * CW]
              for c in range(CH)]
        s1 = _sortnet(ch[:8], _S8)
        s2 = _sortnet(ch[8:], _S8)
        bit = [jnp.maximum(s1[i], s2[7 - i]) for i in range(8)]
        stk = _sortnet(bit, _B8)
        for i in range(TOPK):
            m = jnp.max(stk[0], axis=1, keepdims=True)
            tv_parts[i].append(m)
            if i < TOPK - 1:
                cond = stk[0] == m
                for d in range(7 - i):
                    stk[d] = jnp.where(cond, stk[d + 1], stk[d])

    tv = [jnp.concatenate(p, axis=0) for p in tv_parts]
    kth = tv[-1]
    fam = sum(tv) / TOPK
    exps = [jnp.exp(t - tv[0]) for t in tv]
    z = sum(exps)
    y = sum(ev * t for ev, t in zip(exps, tv)) / z
    g = (fam > THETA).astype(jnp.float32)

    for h in range(2):
        masked_ref[0, :, pl.ds(h * (M // 2), M // 2)] = jnp.where(
            halves[h] >= kth, halves[h], -jnp.inf)

    @pl.when(e == 0)
    def _():
        mf_ref[...] = fam
        w_ref[...] = jnp.zeros((B_TILE, 1), jnp.int32)
        yo_ref[...] = y
        go_ref[...] = g

    @pl.when(e > 0)
    def _():
        wm = mf_ref[...]
        cond = fam > wm
        mf_ref[...] = jnp.where(cond, fam, wm)
        w_ref[...] = jnp.where(cond, e, w_ref[...])
        yo_ref[...] = jnp.where(cond, y, yo_ref[...])
        go_ref[...] = jnp.where(cond, g, go_ref[...])


@jax.jit
def kernel(x, keys, v):
    masked, winner, max_fam, y, g = pl.pallas_call(
        _expert_body,
        grid=(N_EXPERTS,),
        in_specs=[
            pl.BlockSpec((BATCH, D_INPUT), lambda e: (0, 0)),
            pl.BlockSpec((1, M, D_INPUT), lambda e: (e, 0, 0)),
        ],
        out_specs=[
            pl.BlockSpec((1, B_TILE, M), lambda e: (e, 0, 0)),
            pl.BlockSpec((BATCH, 1), lambda e: (0, 0)),
            pl.BlockSpec((BATCH, 1), lambda e: (0, 0)),
            pl.BlockSpec((BATCH, 1), lambda e: (0, 0)),
            pl.BlockSpec((BATCH, 1), lambda e: (0, 0)),
        ],
        out_shape=[
            jax.ShapeDtypeStruct((N_EXPERTS, BATCH, M), jnp.float32),
            jax.ShapeDtypeStruct((BATCH, 1), jnp.int32),
            jax.ShapeDtypeStruct((BATCH, 1), jnp.float32),
            jax.ShapeDtypeStruct((BATCH, 1), jnp.float32),
            jax.ShapeDtypeStruct((BATCH, 1), jnp.float32),
        ],
        scratch_shapes=[
            pltpu.VMEM((BATCH, D_INPUT), jnp.bfloat16),
            pltpu.VMEM((M, D_INPUT), jnp.bfloat16),
        ],
    )(x, keys)

    return (winner.reshape(BATCH), max_fam.reshape(BATCH),
            y.reshape(BATCH), g.reshape(BATCH), masked)
